# baseline (speedup 1.0000x reference)
import functools

import jax
import jax.numpy as jnp
from jax import lax
from jax.experimental import pallas as pl
from jax.experimental.pallas import tpu as pltpu
from jax.experimental.pallas import tpu_sc as plsc

HIDDEN = 1024
EMB = 128
BATCH = 4096
NUM_CAND = 200
LANES = 16


_BM = 512


def _proj_body(x_ref, w_ref, b_ref, o_ref):
    o_ref[...] = (
        jnp.dot(x_ref[...], w_ref[...], preferred_element_type=jnp.float32)
        + b_ref[...]
    )


def _project(pooled_output, W_proj, b_proj):
    return pl.pallas_call(
        _proj_body,
        grid=(BATCH // _BM,),
        in_specs=[
            pl.BlockSpec((_BM, HIDDEN), lambda i: (i, 0)),
            pl.BlockSpec((HIDDEN, EMB), lambda i: (0, 0)),
            pl.BlockSpec((1, EMB), lambda i: (0, 0)),
        ],
        out_specs=pl.BlockSpec((_BM, EMB), lambda i: (i, 0)),
        out_shape=jax.ShapeDtypeStruct((BATCH, EMB), jnp.float32),
    )(pooled_output, W_proj, b_proj.reshape(1, EMB))


_INFO = plsc.get_sparse_core_info()
_NC = _INFO.num_cores
_NS = _INFO.num_subcores
_NW = _NC * _NS
_B_PER = BATCH // _NW

_CHUNK0 = 128
_CHUNK1 = NUM_CAND - _CHUNK0
_CPAD = 208


_NBUF = 3
_OROWS = 32


def _score_body(table_hbm, idx_hbm, text_hbm, out_hbm, idx_v, text_v, out_v,
                emb0_v, emb1_v, emb2_v, sem0, sem1, sem2):
    wid = lax.axis_index("s") * _NC + lax.axis_index("c")
    base = wid * _B_PER
    pltpu.sync_copy(idx_hbm.at[pl.ds(base, _B_PER)], idx_v)
    pltpu.sync_copy(text_hbm.at[pl.ds(base, _B_PER)], text_v)
    lane = lax.iota(jnp.int32, LANES)

    embs = (emb0_v, emb1_v, emb2_v)
    sems = (sem0, sem1, sem2)

    chunks = ((0, 56), (56, 48), (104, 48), (152, 48))

    def fire(b, j):
        for off, n in chunks:
            pltpu.async_copy(
                table_hbm.at[idx_v.at[b, pl.ds(off, n)]],
                embs[j].at[pl.ds(off, n)], sems[j])

    def drain(j):
        for off, n in chunks:
            pltpu.make_async_copy(
                table_hbm.at[pl.ds(0, n)],
                embs[j].at[pl.ds(off, n)], sems[j]).wait()

    def compute(b, j):
        emb_v = embs[j]
        t = [text_v[b, pl.ds(k * LANES, LANES)] for k in range(EMB // LANES)]
        slot = b & (_OROWS - 1)

        def cand_body(cc, inner):
            grp = jnp.zeros((LANES,), jnp.float32)
            for u in range(LANES):
                c = cc * LANES + u
                s = emb_v[c, pl.ds(0, LANES)] * t[0]
                for k in range(1, EMB // LANES):
                    s = s + emb_v[c, pl.ds(k * LANES, LANES)] * t[k]
                grp = jnp.where(lane == u, jnp.sum(s), grp)
            out_v[slot, pl.ds(cc * LANES, LANES)] = grp
            return inner

        lax.fori_loop(0, _CPAD // LANES, cand_body, 0)

    def flush(b):
        pltpu.sync_copy(
            out_v.at[:, pl.ds(0, NUM_CAND)],
            out_hbm.at[pl.ds(base + b - (_OROWS - 1), _OROWS)])

    for j in range(_NBUF):
        fire(j, j)

    def tri_body(p, carry):
        for j in range(_NBUF):
            b = p * _NBUF + j
            drain(j)
            compute(b, j)

            @pl.when(b + _NBUF < _B_PER)
            def _():
                fire(b + _NBUF, j)

            @pl.when((b & (_OROWS - 1)) == _OROWS - 1)
            def _():
                flush(b)
        return carry

    n_tri = (_B_PER - 2) // _NBUF
    lax.fori_loop(0, n_tri, tri_body, 0)
    for b in (_B_PER - 2, _B_PER - 1):
        j = b % _NBUF
        drain(j)
        compute(b, j)
    flush(_B_PER - 1)


_score = functools.partial(
    pl.kernel,
    mesh=plsc.VectorSubcoreMesh(core_axis_name="c", subcore_axis_name="s"),
    compiler_params=pltpu.CompilerParams(
        use_tc_tiling_on_sc=False, needs_layout_passes=False,
        skip_device_barrier=True),
    out_type=jax.ShapeDtypeStruct((BATCH, NUM_CAND), jnp.float32),
    scratch_types=[
        pltpu.VMEM((_B_PER, NUM_CAND), jnp.int32),
        pltpu.VMEM((_B_PER, EMB), jnp.float32),
        pltpu.VMEM((_OROWS, _CPAD), jnp.float32),
        pltpu.VMEM((_CPAD, EMB), jnp.float32),
        pltpu.VMEM((_CPAD, EMB), jnp.float32),
        pltpu.VMEM((_CPAD, EMB), jnp.float32),
        pltpu.SemaphoreType.DMA,
        pltpu.SemaphoreType.DMA,
        pltpu.SemaphoreType.DMA,
    ],
)(_score_body)


@jax.jit
def kernel(pooled_output, candidate_indices, W_proj, b_proj, label_table):
    text_repr = _project(pooled_output, W_proj, b_proj)
    idx = candidate_indices.astype(jnp.int32)
    return _score(label_table, idx, text_repr)


# speedup vs baseline: 1.0094x; 1.0094x over previous
import functools

import jax
import jax.numpy as jnp
from jax import lax
from jax.experimental import pallas as pl
from jax.experimental.pallas import tpu as pltpu
from jax.experimental.pallas import tpu_sc as plsc

HIDDEN = 1024
EMB = 128
BATCH = 4096
NUM_CAND = 200
LANES = 16


_BM = 1024


def _proj_body(x_ref, w_ref, b_ref, o_ref):
    o_ref[...] = (
        jnp.dot(x_ref[...], w_ref[...], preferred_element_type=jnp.float32)
        + b_ref[...]
    )


def _project(pooled_output, W_proj, b_proj):
    return pl.pallas_call(
        _proj_body,
        grid=(BATCH // _BM,),
        in_specs=[
            pl.BlockSpec((_BM, HIDDEN), lambda i: (i, 0)),
            pl.BlockSpec((HIDDEN, EMB), lambda i: (0, 0)),
            pl.BlockSpec((1, EMB), lambda i: (0, 0)),
        ],
        out_specs=pl.BlockSpec((_BM, EMB), lambda i: (i, 0)),
        out_shape=jax.ShapeDtypeStruct((BATCH, EMB), jnp.float32),
    )(pooled_output, W_proj, b_proj.reshape(1, EMB))


_INFO = plsc.get_sparse_core_info()
_NC = _INFO.num_cores
_NS = _INFO.num_subcores
_NW = _NC * _NS
_B_PER = BATCH // _NW

_CHUNK0 = 128
_CHUNK1 = NUM_CAND - _CHUNK0
_CPAD = 208


_NBUF = 3
_OROWS = 32


def _score_body(table_hbm, idx_hbm, text_hbm, out_hbm, idx_v, text_v, out_v,
                emb0_v, emb1_v, emb2_v, sem0, sem1, sem2):
    wid = lax.axis_index("s") * _NC + lax.axis_index("c")
    base = wid * _B_PER
    pltpu.sync_copy(idx_hbm.at[pl.ds(base, _B_PER)], idx_v)
    pltpu.sync_copy(text_hbm.at[pl.ds(base, _B_PER)], text_v)
    lane = lax.iota(jnp.int32, LANES)

    embs = (emb0_v, emb1_v, emb2_v)
    sems = (sem0, sem1, sem2)

    chunks = ((0, 56), (56, 48), (104, 48), (152, 48))

    def fire(b, j):
        for off, n in chunks:
            pltpu.async_copy(
                table_hbm.at[idx_v.at[b, pl.ds(off, n)]],
                embs[j].at[pl.ds(off, n)], sems[j])

    def drain(j):
        for off, n in chunks:
            pltpu.make_async_copy(
                table_hbm.at[pl.ds(0, n)],
                embs[j].at[pl.ds(off, n)], sems[j]).wait()

    def compute(b, j):
        emb_v = embs[j]
        t = [text_v[b, pl.ds(k * LANES, LANES)] for k in range(EMB // LANES)]
        slot = b & (_OROWS - 1)

        def cand_body(cc, inner):
            grp = jnp.zeros((LANES,), jnp.float32)
            for u in range(LANES):
                c = cc * LANES + u
                s = emb_v[c, pl.ds(0, LANES)] * t[0]
                for k in range(1, EMB // LANES):
                    s = s + emb_v[c, pl.ds(k * LANES, LANES)] * t[k]
                grp = jnp.where(lane == u, jnp.sum(s), grp)
            out_v[slot, pl.ds(cc * LANES, LANES)] = grp
            return inner

        lax.fori_loop(0, _CPAD // LANES, cand_body, 0)

    def flush(b):
        pltpu.sync_copy(
            out_v.at[:, pl.ds(0, NUM_CAND)],
            out_hbm.at[pl.ds(base + b - (_OROWS - 1), _OROWS)])

    for j in range(_NBUF):
        fire(j, j)

    def tri_body(p, carry):
        for j in range(_NBUF):
            b = p * _NBUF + j
            drain(j)
            compute(b, j)

            @pl.when(b + _NBUF < _B_PER)
            def _():
                fire(b + _NBUF, j)

            @pl.when((b & (_OROWS - 1)) == _OROWS - 1)
            def _():
                flush(b)
        return carry

    n_tri = (_B_PER - 2) // _NBUF
    lax.fori_loop(0, n_tri, tri_body, 0)
    for b in (_B_PER - 2, _B_PER - 1):
        j = b % _NBUF
        drain(j)
        compute(b, j)
    flush(_B_PER - 1)


_score = functools.partial(
    pl.kernel,
    mesh=plsc.VectorSubcoreMesh(core_axis_name="c", subcore_axis_name="s"),
    compiler_params=pltpu.CompilerParams(
        use_tc_tiling_on_sc=False, needs_layout_passes=False),
    out_type=jax.ShapeDtypeStruct((BATCH, NUM_CAND), jnp.float32),
    scratch_types=[
        pltpu.VMEM((_B_PER, NUM_CAND), jnp.int32),
        pltpu.VMEM((_B_PER, EMB), jnp.float32),
        pltpu.VMEM((_OROWS, _CPAD), jnp.float32),
        pltpu.VMEM((_CPAD, EMB), jnp.float32),
        pltpu.VMEM((_CPAD, EMB), jnp.float32),
        pltpu.VMEM((_CPAD, EMB), jnp.float32),
        pltpu.SemaphoreType.DMA,
        pltpu.SemaphoreType.DMA,
        pltpu.SemaphoreType.DMA,
    ],
)(_score_body)


@jax.jit
def kernel(pooled_output, candidate_indices, W_proj, b_proj, label_table):
    text_repr = _project(pooled_output, W_proj, b_proj)
    idx = candidate_indices.astype(jnp.int32)
    return _score(label_table, idx, text_repr)
